# Initial kernel scaffold; baseline (speedup 1.0000x reference)
import jax
import jax.numpy as jnp
from jax.experimental import pallas as pl


def kernel(inputs, tables, w0, w, V, W1, b1, W2, b2, W3, b3, Wo, bo):
    raise NotImplementedError("write your pallas kernel here")


# speedup vs baseline: 1.9756x; 1.9756x over previous
import functools

import jax
import jax.numpy as jnp
from jax import lax
from jax.experimental import pallas as pl
from jax.experimental.pallas import tpu as pltpu
from jax.experimental.pallas import tpu_sc as plsc

B = 1024
DENSE_DIM = 13
SPARSE_DIM = 26
VOCAB = 1000
EMB = 64
FN = DENSE_DIM + SPARSE_DIM * EMB

NC, NS = 2, 16
NW = NC * NS
ROWS = B * SPARSE_DIM
ROWS_PER_W = ROWS // NW
CHUNK = 104
NCHUNK = ROWS_PER_W // CHUNK

_sc_mesh = plsc.VectorSubcoreMesh(
    core_axis_name="c", subcore_axis_name="s", num_cores=NC, num_subcores=NS)


@functools.partial(
    pl.kernel,
    mesh=_sc_mesh,
    out_type=jax.ShapeDtypeStruct((ROWS, EMB), jnp.float32),
    scratch_types=[
        pltpu.VMEM((NCHUNK, CHUNK), jnp.int32),
        pltpu.VMEM((ROWS_PER_W, EMB), jnp.float32),
        pltpu.SemaphoreType.DMA,
    ],
    compiler_params=pltpu.CompilerParams(use_tc_tiling_on_sc=False),
)
def _sc_gather(table_hbm, idx_hbm, out_hbm, idx_v, rows_v, sem):
    wid = lax.axis_index("s") * NC + lax.axis_index("c")
    pltpu.sync_copy(idx_hbm.at[pl.ds(wid * NCHUNK, NCHUNK)], idx_v)
    copies = []
    for j in range(NCHUNK):
        copies.append(
            pltpu.async_copy(
                table_hbm.at[idx_v.at[j]],
                rows_v.at[pl.ds(j * CHUNK, CHUNK)],
                sem,
            ))
    for c in copies:
        c.wait()
    pltpu.sync_copy(rows_v, out_hbm.at[pl.ds(wid * ROWS_PER_W, ROWS_PER_W)])


def _tc_body(x_ref, w0_ref, w_ref, V_ref, W1_ref, b1_ref, W2_ref, b2_ref,
             W3_ref, b3_ref, Wo_ref, bo_ref, out_ref):
    x = x_ref[...]
    V = V_ref[...]
    xv = jnp.dot(x, V, preferred_element_type=jnp.float32)
    s1 = jnp.sum(xv * xv, axis=1, keepdims=True)
    v2s = jnp.sum(V * V, axis=1, keepdims=True)
    s2 = jnp.dot(x * x, v2s, preferred_element_type=jnp.float32)
    lin = jnp.dot(x, w_ref[...], preferred_element_type=jnp.float32)
    fm = w0_ref[0, 0] + lin + 0.5 * (s1 - s2)
    h = jnp.maximum(
        jnp.dot(x, W1_ref[...], preferred_element_type=jnp.float32)
        + b1_ref[...], 0.0)
    h = jnp.maximum(
        jnp.dot(h, W2_ref[...], preferred_element_type=jnp.float32)
        + b2_ref[...], 0.0)
    h = jnp.maximum(
        jnp.dot(h, W3_ref[...], preferred_element_type=jnp.float32)
        + b3_ref[...], 0.0)
    deep = jnp.dot(h, Wo_ref[...], preferred_element_type=jnp.float32) + bo_ref[0, 0]
    out_ref[...] = jax.nn.sigmoid(0.5 * (fm + deep))


_BLK = 128
_H1, _H2, _H3 = 1024, 512, 256


def _full(shape):
    return pl.BlockSpec(shape, lambda i: (0, 0))


_tc_call = pl.pallas_call(
    _tc_body,
    grid=(B // _BLK,),
    in_specs=[
        pl.BlockSpec((_BLK, FN), lambda i: (i, 0)),
        _full((1, 1)),
        _full((FN, 1)),
        _full((FN, 64)),
        _full((FN, _H1)),
        _full((1, _H1)),
        _full((_H1, _H2)),
        _full((1, _H2)),
        _full((_H2, _H3)),
        _full((1, _H3)),
        _full((_H3, 1)),
        _full((1, 1)),
    ],
    out_specs=pl.BlockSpec((_BLK, 1), lambda i: (i, 0)),
    out_shape=jax.ShapeDtypeStruct((B, 1), jnp.float32),
    compiler_params=pltpu.CompilerParams(
        dimension_semantics=("arbitrary",)),
)


def kernel(inputs, tables, w0, w, V, W1, b1, W2, b2, W3, b3, Wo, bo):
    dense = inputs[:, :DENSE_DIM]
    sparse_idx = inputs[:, DENSE_DIM:].astype(jnp.int32)
    flat_idx = (sparse_idx
                + jnp.arange(SPARSE_DIM, dtype=jnp.int32)[None, :] * VOCAB)
    idx2d = flat_idx.reshape(NW * NCHUNK, CHUNK)
    table2d = tables.reshape(SPARSE_DIM * VOCAB, EMB)
    emb = _sc_gather(table2d, idx2d)
    x = jnp.concatenate([dense, emb.reshape(B, SPARSE_DIM * EMB)], axis=1)
    out = _tc_call(x, w0.reshape(1, 1), w, V, W1, b1.reshape(1, _H1),
                   W2, b2.reshape(1, _H2), W3, b3.reshape(1, _H3),
                   Wo, bo.reshape(1, 1))
    return out


# speedup vs baseline: 2.0268x; 1.0259x over previous
import functools

import jax
import jax.numpy as jnp
from jax import lax
from jax.experimental import pallas as pl
from jax.experimental.pallas import tpu as pltpu
from jax.experimental.pallas import tpu_sc as plsc

B = 1024
DENSE_DIM = 13
SPARSE_DIM = 26
VOCAB = 1000
EMB = 64
FN = DENSE_DIM + SPARSE_DIM * EMB

NC, NS = 2, 16
NW = NC * NS
ROWS = B * SPARSE_DIM
ROWS_PER_W = ROWS // NW
CHUNK = 104
NCHUNK = ROWS_PER_W // CHUNK

_sc_mesh = plsc.VectorSubcoreMesh(
    core_axis_name="c", subcore_axis_name="s", num_cores=NC, num_subcores=NS)


@functools.partial(
    pl.kernel,
    mesh=_sc_mesh,
    out_type=jax.ShapeDtypeStruct((ROWS, EMB), jnp.float32),
    scratch_types=[
        pltpu.VMEM((NCHUNK, CHUNK), jnp.int32),
        pltpu.VMEM((ROWS_PER_W, EMB), jnp.float32),
        pltpu.SemaphoreType.DMA,
    ],
    compiler_params=pltpu.CompilerParams(use_tc_tiling_on_sc=False),
)
def _sc_gather(table_hbm, idx_hbm, out_hbm, idx_v, rows_v, sem):
    wid = lax.axis_index("s") * NC + lax.axis_index("c")
    pltpu.sync_copy(idx_hbm.at[pl.ds(wid * NCHUNK, NCHUNK)], idx_v)
    copies = []
    for j in range(NCHUNK):
        copies.append(
            pltpu.async_copy(
                table_hbm.at[idx_v.at[j]],
                rows_v.at[pl.ds(j * CHUNK, CHUNK)],
                sem,
            ))
    for c in copies:
        c.wait()
    pltpu.sync_copy(rows_v, out_hbm.at[pl.ds(wid * ROWS_PER_W, ROWS_PER_W)])


def _tc_body(x_ref, w0_ref, w_ref, V_ref, W1_ref, b1_ref, W2_ref, b2_ref,
             W3_ref, b3_ref, Wo_ref, bo_ref, out_ref):
    x = x_ref[...]
    V = V_ref[...]
    xv = jnp.dot(x, V, preferred_element_type=jnp.float32)
    s1 = jnp.sum(xv * xv, axis=1, keepdims=True)
    v2s = jnp.sum(V * V, axis=1, keepdims=True)
    s2 = jnp.dot(x * x, v2s, preferred_element_type=jnp.float32)
    lin = jnp.dot(x, w_ref[...], preferred_element_type=jnp.float32)
    fm = w0_ref[0, 0] + lin + 0.5 * (s1 - s2)
    xb = x.astype(jnp.bfloat16)
    h = jnp.maximum(
        jnp.dot(xb, W1_ref[...].astype(jnp.bfloat16),
                preferred_element_type=jnp.float32) + b1_ref[...], 0.0)
    h = jnp.maximum(
        jnp.dot(h.astype(jnp.bfloat16), W2_ref[...].astype(jnp.bfloat16),
                preferred_element_type=jnp.float32) + b2_ref[...], 0.0)
    h = jnp.maximum(
        jnp.dot(h.astype(jnp.bfloat16), W3_ref[...].astype(jnp.bfloat16),
                preferred_element_type=jnp.float32) + b3_ref[...], 0.0)
    deep = jnp.dot(h, Wo_ref[...], preferred_element_type=jnp.float32) + bo_ref[0, 0]
    out_ref[...] = jax.nn.sigmoid(0.5 * (fm + deep))


_BLK = 1024
_H1, _H2, _H3 = 1024, 512, 256


def _full(shape):
    return pl.BlockSpec(shape, lambda i: (0, 0))


_tc_call = pl.pallas_call(
    _tc_body,
    grid=(B // _BLK,),
    in_specs=[
        pl.BlockSpec((_BLK, FN), lambda i: (i, 0)),
        _full((1, 1)),
        _full((FN, 1)),
        _full((FN, 64)),
        _full((FN, _H1)),
        _full((1, _H1)),
        _full((_H1, _H2)),
        _full((1, _H2)),
        _full((_H2, _H3)),
        _full((1, _H3)),
        _full((_H3, 1)),
        _full((1, 1)),
    ],
    out_specs=pl.BlockSpec((_BLK, 1), lambda i: (i, 0)),
    out_shape=jax.ShapeDtypeStruct((B, 1), jnp.float32),
    compiler_params=pltpu.CompilerParams(
        dimension_semantics=("arbitrary",)),
)


def kernel(inputs, tables, w0, w, V, W1, b1, W2, b2, W3, b3, Wo, bo):
    dense = inputs[:, :DENSE_DIM]
    sparse_idx = inputs[:, DENSE_DIM:].astype(jnp.int32)
    flat_idx = (sparse_idx
                + jnp.arange(SPARSE_DIM, dtype=jnp.int32)[None, :] * VOCAB)
    idx2d = flat_idx.reshape(NW * NCHUNK, CHUNK)
    table2d = tables.reshape(SPARSE_DIM * VOCAB, EMB)
    emb = _sc_gather(table2d, idx2d)
    x = jnp.concatenate([dense, emb.reshape(B, SPARSE_DIM * EMB)], axis=1)
    out = _tc_call(x, w0.reshape(1, 1), w, V, W1, b1.reshape(1, _H1),
                   W2, b2.reshape(1, _H2), W3, b3.reshape(1, _H3),
                   Wo, bo.reshape(1, 1))
    return out
